# baseline (speedup 1.0000x reference)
import functools

import jax
import jax.numpy as jnp
from jax import lax
from jax.experimental import pallas as pl
from jax.experimental.pallas import tpu as pltpu
from jax.experimental.pallas import tpu_sc as plsc


def _normalize_transpose_body(x_ref, o_ref):
    x = x_ref[0]
    C = x.shape[0]
    s = jnp.sum(x * x, axis=0, keepdims=True)
    inv = 1.0 / (jnp.sqrt(s) + 1e-7)
    y = (x * inv).astype(jnp.bfloat16)
    row = lax.broadcasted_iota(jnp.int32, (C, C), 0)
    col = lax.broadcasted_iota(jnp.int32, (C, C), 1)
    eye = (row == col).astype(jnp.bfloat16)
    yt = lax.dot_general(
        y, eye, (((0,), (0,)), ((), ())),
        preferred_element_type=jnp.float32,
    )

    def bf16_bits(v):
        i = lax.bitcast_convert_type(v, jnp.int32)
        r = ((i >> 16) & 1) + 0x7FFF
        return ((i + r) >> 16) & 0xFFFF

    lo = bf16_bits(yt[:, : C // 2])
    hi = bf16_bits(yt[:, C // 2:])
    o_ref[0] = lo | (hi << 16)


def _build_table(feats_2d, C, HW, HWB):
    FB = feats_2d.shape[0]
    return pl.pallas_call(
        _normalize_transpose_body,
        grid=(FB, HW // HWB),
        in_specs=[pl.BlockSpec((1, C, HWB), lambda i, j: (i, 0, j))],
        out_specs=pl.BlockSpec((1, HWB, C // 2), lambda i, j: (i, j, 0)),
        out_shape=jax.ShapeDtypeStruct((FB, HW, C // 2), jnp.int32),
    )(feats_2d)


def _sc_gather(table, idx, R, C, chunk):
    NC, NS = 2, 16
    NW = NC * NS
    r_per_w = R // NW
    n_chunks = r_per_w // chunk

    mesh = plsc.VectorSubcoreMesh(core_axis_name="c", subcore_axis_name="s")

    @functools.partial(
        pl.kernel,
        mesh=mesh,
        out_type=jax.ShapeDtypeStruct((R, C), jnp.int32),
        scratch_types=[
            pltpu.VMEM((chunk,), jnp.int32),
            pltpu.VMEM((chunk, C), jnp.int32),
            pltpu.SemaphoreType.DMA,
        ],
    )
    def gather_kernel(table_hbm, idx_hbm, out_hbm, idx_v, rows_v, sem):
        wid = lax.axis_index("s") * NC + lax.axis_index("c")
        base = wid * r_per_w

        def body(g, carry):
            off = base + g * chunk
            pltpu.sync_copy(idx_hbm.at[pl.ds(off, chunk)], idx_v)
            pltpu.async_copy(table_hbm.at[idx_v], rows_v, sem).wait()
            pltpu.sync_copy(rows_v, out_hbm.at[pl.ds(off, chunk)])
            return carry

        lax.fori_loop(0, n_chunks, body, 0)

    return gather_kernel(table, idx)


def _widen_body(x_ref, o_ref):
    x = x_ref[...]
    CW = x.shape[1]
    lo = lax.bitcast_convert_type(x << 16, jnp.float32)
    hi = lax.bitcast_convert_type(x & jnp.int32(-65536), jnp.float32)
    o_ref[:, :CW] = lo
    o_ref[:, CW:] = hi


def _widen(x, R, C, RB):
    return pl.pallas_call(
        _widen_body,
        grid=(R // RB,),
        in_specs=[pl.BlockSpec((RB, C // 2), lambda i: (i, 0))],
        out_specs=pl.BlockSpec((RB, C), lambda i: (i, 0)),
        out_shape=jax.ShapeDtypeStruct((R, C), jnp.float32),
    )(x)


def kernel(feats, num_patches, patch_ids):
    F_, B, C, H, W = feats.shape
    HW = H * W
    FB = F_ * B
    P = patch_ids.shape[-1]
    R = FB * P

    table = _build_table(feats.reshape(FB, C, HW), C, HW, 2048)
    table = table.reshape(FB * HW, C // 2)

    row_off = (jnp.arange(FB, dtype=jnp.int32) * HW)[:, None]
    idx = (patch_ids.reshape(FB, P) + row_off).reshape(R)

    out_bf = _sc_gather(table, idx, R, C // 2, 128)
    out = _widen(out_bf, R, C, 4096)
    return out.reshape(F_, B * P, C)
